# baseline (speedup 1.0000x reference)
import jax
import jax.numpy as jnp
from jax import lax
from jax.experimental import pallas as pl
from jax.experimental.pallas import tpu as pltpu
from jax.experimental.pallas import tpu_sc as plsc

MAXSPAN = 255
QLEN = 2048
KLEN = 2048
DEPTH = 64
EXT = 4160
LO_PAD = 1793
HI_START = 2302
WROWS = KLEN * DEPTH // 128
NPH = 16
PH_ROWS = 2048
NBUF = 8

NCORES = 2
NSUB = 16
NWORKERS = NCORES * NSUB
ROWS_PER_W = (QLEN // 2) // NWORKERS
RING = 8


def _build_ext_kernel(emb_ref, ext_ref):
    ext_ref[0:LO_PAD, :] = jnp.broadcast_to(emb_ref[0:1, :], (LO_PAD, DEPTH))
    ext_ref[LO_PAD:HI_START, :] = emb_ref[:, :]
    ext_ref[HI_START:EXT, :] = jnp.broadcast_to(
        emb_ref[508:509, :], (EXT - HI_START, DEPTH))


def _tc_even_kernel(ext_a, ext_b, out_ref, phases, stage_sem, sems):
    def stage_copy(p):
        src = (ext_a.at[pl.ds(p // 2, PH_ROWS), :] if p % 2 == 0
               else ext_b.at[pl.ds((p - 1) // 2, PH_ROWS), :])
        return pltpu.make_async_copy(src, phases.at[p], stage_sem)

    for p in range(NPH):
        stage_copy(p).start()
    for p in range(NPH):
        stage_copy(p).wait()

    def row_copy(i, b):
        s = QLEN - 1 - 2 * i
        p = lax.rem(s, NPH)
        t = lax.div(s, NPH)
        return pltpu.make_async_copy(
            phases.at[p].at[pl.ds(8 * t, WROWS), :],
            out_ref.at[i],
            sems.at[b])

    nrows = QLEN // 2

    def loop(g, _):
        for b in range(NBUF):
            i = g * NBUF + b

            @pl.when(g >= 1)
            def _():
                row_copy(i - NBUF, b).wait()

            row_copy(i, b).start()
        return ()

    lax.fori_loop(0, nrows // NBUF, loop, ())

    for b in range(NBUF):
        row_copy(nrows - NBUF + b, b).wait()


def _sc_odd_kernel(ext_a, out_hbm, ext_sp, sem):
    c = lax.axis_index("c")
    s = lax.axis_index("s")

    @pl.when(s == 0)
    def _():
        pltpu.sync_copy(ext_a, ext_sp)

    plsc.subcore_barrier()

    base = (c * NSUB + s) * ROWS_PER_W

    def row_copy(i):
        j = base + i
        return pltpu.make_async_copy(
            ext_sp.at[pl.ds(WROWS - 1 - j, WROWS), :],
            out_hbm.at[j],
            sem)

    def fire(i, _):
        @pl.when(i >= RING)
        def _():
            row_copy(i - RING).wait()
        row_copy(i).start()
        return ()

    lax.fori_loop(0, ROWS_PER_W, fire, ())

    def drain(i, _):
        row_copy(ROWS_PER_W - RING + i).wait()
        return ()

    lax.fori_loop(0, RING, drain, ())


def kernel(inputs, embeddings):
    del inputs
    ext = pl.pallas_call(
        _build_ext_kernel,
        out_shape=jax.ShapeDtypeStruct((EXT, DEPTH), jnp.float32),
    )(embeddings)

    ext_flat = ext.reshape(-1)
    ext_a = ext_flat.reshape(EXT * DEPTH // 128, 128)
    ext_b = ext_flat[64:64 + (EXT * DEPTH // 128 - 1) * 128].reshape(
        EXT * DEPTH // 128 - 1, 128)

    w_even = pl.pallas_call(
        _tc_even_kernel,
        in_specs=[pl.BlockSpec(memory_space=pl.ANY),
                  pl.BlockSpec(memory_space=pl.ANY)],
        out_specs=pl.BlockSpec(memory_space=pl.ANY),
        out_shape=jax.ShapeDtypeStruct((QLEN // 2, WROWS, 128), jnp.float32),
        scratch_shapes=[
            pltpu.VMEM((NPH, PH_ROWS, 128), jnp.float32),
            pltpu.SemaphoreType.DMA,
            pltpu.SemaphoreType.DMA((NBUF,)),
        ],
    )(ext_a, ext_b)

    sc_expand = pl.kernel(
        _sc_odd_kernel,
        out_type=jax.ShapeDtypeStruct((QLEN // 2, WROWS, 128), jnp.float32),
        mesh=plsc.VectorSubcoreMesh(
            core_axis_name="c", subcore_axis_name="s"),
        scratch_types=[
            pltpu.VMEM_SHARED((EXT * DEPTH // 128, 128), jnp.float32),
            pltpu.SemaphoreType.DMA,
        ],
    )
    w_odd = sc_expand(ext_a)

    out = jnp.stack([w_even, w_odd], axis=1)
    return out.reshape(QLEN, KLEN, DEPTH)


# speedup vs baseline: 1.8257x; 1.8257x over previous
import jax
import jax.numpy as jnp
from jax import lax
from jax.experimental import pallas as pl
from jax.experimental.pallas import tpu as pltpu
from jax.experimental.pallas import tpu_sc as plsc

MAXSPAN = 255
QLEN = 2048
KLEN = 2048
DEPTH = 64
EXT = 4160
LO_PAD = 1793
HI_START = 2302
WROWS = KLEN * DEPTH // 128

NCORES = 2
NSUB = 16
NWORKERS = NCORES * NSUB
ROWS_PER_W = QLEN // NWORKERS
RING = 8


def _build_ext_kernel(emb_ref, ext_ref):
    ext_ref[0:LO_PAD, :] = jnp.broadcast_to(emb_ref[0:1, :], (LO_PAD, DEPTH))
    ext_ref[LO_PAD:HI_START, :] = emb_ref[:, :]
    ext_ref[HI_START:EXT, :] = jnp.broadcast_to(
        emb_ref[508:509, :], (EXT - HI_START, DEPTH))


def _sc_expand(ext_a, ext_b, out_hbm, sp0, sp1, sem):
    c = lax.axis_index("c")
    s = lax.axis_index("s")

    @pl.when(s == 0)
    def _():
        pltpu.sync_copy(ext_a, sp0)

    @pl.when(s == 1)
    def _():
        pltpu.sync_copy(ext_b, sp1)

    plsc.subcore_barrier()

    base = (c * NSUB + s) * ROWS_PER_W

    def row_even(j):
        q = base + 2 * j
        return pltpu.make_async_copy(
            sp1.at[pl.ds(WROWS - 1 - lax.div(q, 2), WROWS), :],
            out_hbm.at[q],
            sem)

    def row_odd(j):
        q = base + 2 * j + 1
        return pltpu.make_async_copy(
            sp0.at[pl.ds(WROWS - 1 - lax.div(q - 1, 2), WROWS), :],
            out_hbm.at[q],
            sem)

    npairs = ROWS_PER_W // 2

    def fire(j, _):
        @pl.when(j >= RING)
        def _():
            row_even(j - RING).wait()
            row_odd(j - RING).wait()
        row_even(j).start()
        row_odd(j).start()
        return ()

    lax.fori_loop(0, npairs, fire, ())

    def drain(j, _):
        row_even(npairs - RING + j).wait()
        row_odd(npairs - RING + j).wait()
        return ()

    lax.fori_loop(0, RING, drain, ())


def kernel(inputs, embeddings):
    del inputs
    ext = pl.pallas_call(
        _build_ext_kernel,
        out_shape=jax.ShapeDtypeStruct((EXT, DEPTH), jnp.float32),
    )(embeddings)
    ext_flat = ext.reshape(-1)
    nrows_a = EXT * DEPTH // 128
    ext_a = ext_flat.reshape(nrows_a, 128)
    ext_b = ext_flat[64:64 + (nrows_a - 1) * 128].reshape(nrows_a - 1, 128)

    sc_expand = pl.kernel(
        _sc_expand,
        out_type=jax.ShapeDtypeStruct((QLEN, WROWS, 128), jnp.float32),
        mesh=plsc.VectorSubcoreMesh(
            core_axis_name="c", subcore_axis_name="s"),
        scratch_types=[
            pltpu.VMEM_SHARED((nrows_a, 128), jnp.float32),
            pltpu.VMEM_SHARED((nrows_a - 1, 128), jnp.float32),
            pltpu.SemaphoreType.DMA,
        ],
    )
    out = sc_expand(ext_a, ext_b)
    return out.reshape(QLEN, KLEN, DEPTH)
